# baseline (speedup 1.0000x reference)
import functools

import jax
import jax.numpy as jnp
from jax import lax
from jax.experimental import pallas as pl
from jax.experimental.pallas import tpu as pltpu, tpu_sc as plsc

T = 2048
D = 1024
F = 2048
E = 8
K = 2
S = T * K
TM = 256
R = S // TM + E - 1
P = R * TM

NW = 32
PG = 6144
CH_D = 6
CH_C = 4


def _routing_meta(gating_output):
  probs = jax.nn.softmax(gating_output.astype(jnp.float32), axis=-1)
  topw, topi = lax.top_k(probs, K)
  topw = topw / jnp.sum(topw, axis=-1, keepdims=True)

  e_flat = topi.reshape(-1).astype(jnp.int32)
  w_flat = topw.reshape(-1)
  t_flat = (jnp.arange(S, dtype=jnp.int32) // K)

  onehot = (e_flat[:, None] == jnp.arange(E, dtype=jnp.int32)[None, :])
  onehot = onehot.astype(jnp.int32)
  cum = jnp.cumsum(onehot, axis=0)
  rank = jnp.take_along_axis(cum - onehot, e_flat[:, None], axis=1)[:, 0]
  sizes = cum[-1]
  tiles_e = (sizes + TM - 1) // TM
  tile_end = jnp.cumsum(tiles_e)
  pad_start = (tile_end - tiles_e) * TM

  dest = (pad_start[e_flat] + rank).astype(jnp.int32)

  used_tiles = tile_end[-1]
  tidx = jnp.arange(R, dtype=jnp.int32)
  tile_e = jnp.sum(tidx[:, None] >= tile_end[None, :], axis=-1)
  tile_used = (tidx < used_tiles).astype(jnp.int32)
  last_e = jnp.sum(used_tiles - 1 >= tile_end, axis=-1)
  tile_e = jnp.where(tile_used == 1, tile_e, last_e).astype(jnp.int32)

  tok_pad = jnp.zeros((PG,), jnp.int32).at[dest].set(t_flat)
  w_pad = jnp.zeros((P,), jnp.float32).at[dest].set(w_flat).reshape(P, 1)
  return dest, tok_pad, w_pad, tile_e, tile_used


def _sc_row_gather(table, idx, n_out, chunks):
  rows_w = n_out // NW
  rows_ch = rows_w // chunks
  tail = table.shape[1:]
  mesh = plsc.VectorSubcoreMesh(core_axis_name="c", subcore_axis_name="s")

  @functools.partial(
      pl.kernel,
      out_type=jax.ShapeDtypeStruct((n_out,) + tail, table.dtype),
      mesh=mesh,
      scratch_types=[
          pltpu.VMEM((rows_w,), jnp.int32),
          pltpu.VMEM((rows_ch,) + tail, table.dtype),
          pltpu.VMEM((rows_ch,) + tail, table.dtype),
          pltpu.SemaphoreType.DMA,
          pltpu.SemaphoreType.DMA,
      ],
  )
  def gather_k(table_hbm, idx_hbm, out_hbm, idx_v, rows_a, rows_b, sem_a, sem_b):
    wid = lax.axis_index("s") * 2 + lax.axis_index("c")
    base = wid * rows_w
    pltpu.sync_copy(idx_hbm.at[pl.ds(base, rows_w)], idx_v)
    bufs = [(rows_a, sem_a), (rows_b, sem_b)]
    cps = []
    for j in range(chunks):
      r, s = bufs[j % 2]
      cps.append(pltpu.async_copy(
          table_hbm.at[idx_v.at[pl.ds(j * rows_ch, rows_ch)]], r, s))
      if j >= 1:
        pr, _ = bufs[(j - 1) % 2]
        cps[j - 1].wait()
        pltpu.sync_copy(pr, out_hbm.at[pl.ds(base + (j - 1) * rows_ch, rows_ch)])
    cps[-1].wait()
    pltpu.sync_copy(bufs[(chunks - 1) % 2][0],
                    out_hbm.at[pl.ds(base + (chunks - 1) * rows_ch, rows_ch)])

  return gather_k(table, idx)


def _gemm_body(te_ref, tu_ref, xs_ref, w_ref, g_ref, u_ref, d_ref, y_ref):
  i = pl.program_id(0)

  @pl.when(tu_ref[i] == 1)
  def _():
    xb = xs_ref[...].astype(jnp.float32)
    dn = (((1,), (1,)), ((), ()))
    g = lax.dot_general(xb, g_ref[0], dn, preferred_element_type=jnp.float32)
    u = lax.dot_general(xb, u_ref[0], dn, preferred_element_type=jnp.float32)
    h = g * jax.nn.sigmoid(g) * u
    y = lax.dot_general(h, d_ref[0], dn, preferred_element_type=jnp.float32)
    y_ref[...] = y * w_ref[...]


def _grouped_gemm(xs, w_pad, gate_proj, up_proj, down_proj, tile_e, tile_used):
  grid_spec = pltpu.PrefetchScalarGridSpec(
      num_scalar_prefetch=2,
      grid=(R,),
      in_specs=[
          pl.BlockSpec((TM, D), lambda i, te, tu: (i, 0)),
          pl.BlockSpec((TM, 1), lambda i, te, tu: (i, 0)),
          pl.BlockSpec((1, F, D), lambda i, te, tu: (te[i], 0, 0)),
          pl.BlockSpec((1, F, D), lambda i, te, tu: (te[i], 0, 0)),
          pl.BlockSpec((1, D, F), lambda i, te, tu: (te[i], 0, 0)),
      ],
      out_specs=pl.BlockSpec((TM, D), lambda i, te, tu: (i, 0)),
  )
  return pl.pallas_call(
      _gemm_body,
      grid_spec=grid_spec,
      out_shape=jax.ShapeDtypeStruct((P, D), jnp.float32),
  )(tile_e, tile_used, xs, w_pad, gate_proj, up_proj, down_proj)


def _pair_sum_body(y2_ref, o_ref):
  o_ref[...] = y2_ref[:, :D] + y2_ref[:, D:]


def _pair_sum(y2):
  return pl.pallas_call(
      _pair_sum_body,
      grid=(T // 256,),
      in_specs=[pl.BlockSpec((256, K * D), lambda i: (i, 0))],
      out_specs=pl.BlockSpec((256, D), lambda i: (i, 0)),
      out_shape=jax.ShapeDtypeStruct((T, D), jnp.float32),
  )(y2)


def _sc_combine(y, dest):
  tok_w = T // NW
  tok_ch = 16
  n_ch = tok_w // tok_ch
  mesh = plsc.VectorSubcoreMesh(core_axis_name="c", subcore_axis_name="s")

  @functools.partial(
      pl.kernel,
      out_type=jax.ShapeDtypeStruct((T, 8, 128), jnp.float32),
      mesh=mesh,
      scratch_types=[
          pltpu.VMEM((K * tok_w,), jnp.int32),
          pltpu.VMEM((K * tok_ch, D), jnp.float32),
          pltpu.VMEM((K * tok_ch, D), jnp.float32),
          pltpu.VMEM((tok_ch, 8, 128), jnp.float32),
          pltpu.VMEM((tok_ch, 8, 128), jnp.float32),
          pltpu.SemaphoreType.DMA,
          pltpu.SemaphoreType.DMA,
          pltpu.SemaphoreType.DMA,
          pltpu.SemaphoreType.DMA,
      ],
  )
  def comb_k(y_hbm, dest_hbm, out_hbm, idx_v, ra, rb, oa, ob,
             gs_a, gs_b, ws_a, ws_b):
    wid = lax.axis_index("s") * 2 + lax.axis_index("c")
    base_s = wid * (K * tok_w)
    base_t = wid * tok_w
    pltpu.sync_copy(dest_hbm.at[pl.ds(base_s, K * tok_w)], idx_v)
    bufs = [(ra, gs_a, oa, ws_a), (rb, gs_b, ob, ws_b)]
    gat = [None] * n_ch
    wr = [None] * n_ch

    def process(j):
      r, _, obuf, ws = bufs[j % 2]
      gat[j].wait()

      def body(tk, c):
        for s in range(8):
          for q in range(8):
            off = s * 128 + q * 16
            a = r[2 * tk, pl.ds(off, 16)] + r[2 * tk + 1, pl.ds(off, 16)]
            obuf[tk, s, pl.ds(q * 16, 16)] = a
        return c

      lax.fori_loop(0, tok_ch, body, 0)
      wr[j] = pltpu.async_copy(
          obuf, out_hbm.at[pl.ds(base_t + j * tok_ch, tok_ch)], ws)

    for j in range(n_ch):
      r, gs, _, _ = bufs[j % 2]
      if j >= 2:
        wr[j - 2].wait()
      gat[j] = pltpu.async_copy(
          y_hbm.at[idx_v.at[pl.ds(j * K * tok_ch, K * tok_ch)]], r, gs)
      if j >= 1:
        process(j - 1)
    process(n_ch - 1)
    wr[n_ch - 2].wait()
    wr[n_ch - 1].wait()

  return comb_k(y, dest)


def kernel(x, gating_output, gate_proj, up_proj, down_proj):
  dest, tok_pad, w_pad, tile_e, tile_used = _routing_meta(gating_output)
  x_i = lax.bitcast_convert_type(
      x.astype(jnp.bfloat16).reshape(T, D // 2, 2), jnp.int32)
  x3 = x_i.reshape(T, 4, 128)
  xs_i = _sc_row_gather(x3, tok_pad, PG, CH_D)
  xs = lax.bitcast_convert_type(xs_i, jnp.bfloat16).reshape(PG, D)
  y = _grouped_gemm(xs, w_pad, gate_proj, up_proj, down_proj,
                    tile_e, tile_used)
  y2 = _sc_row_gather(y, dest, S, CH_C)
  return _pair_sum(y2.reshape(T, K * D))


# speedup vs baseline: 1.3495x; 1.3495x over previous
import functools

import jax
import jax.numpy as jnp
from jax import lax
from jax.experimental import pallas as pl
from jax.experimental.pallas import tpu as pltpu, tpu_sc as plsc

T = 2048
D = 1024
F = 2048
E = 8
K = 2
S = T * K
TM = 256
R = S // TM + E - 1
P = R * TM

NW = 32
PG = 6144
CH_D = 6
CH_C = 4


def _routing_meta(gating_output):
  probs = jax.nn.softmax(gating_output.astype(jnp.float32), axis=-1)
  topw, topi = lax.top_k(probs, K)
  topw = topw / jnp.sum(topw, axis=-1, keepdims=True)

  e_flat = topi.reshape(-1).astype(jnp.int32)
  w_flat = topw.reshape(-1)
  t_flat = (jnp.arange(S, dtype=jnp.int32) // K)

  onehot = (e_flat[:, None] == jnp.arange(E, dtype=jnp.int32)[None, :])
  onehot = onehot.astype(jnp.int32)
  cum = jnp.cumsum(onehot, axis=0)
  rank = jnp.take_along_axis(cum - onehot, e_flat[:, None], axis=1)[:, 0]
  sizes = cum[-1]
  tiles_e = (sizes + TM - 1) // TM
  tile_end = jnp.cumsum(tiles_e)
  pad_start = (tile_end - tiles_e) * TM

  dest = (pad_start[e_flat] + rank).astype(jnp.int32)

  used_tiles = tile_end[-1]
  tidx = jnp.arange(R, dtype=jnp.int32)
  tile_e = jnp.sum(tidx[:, None] >= tile_end[None, :], axis=-1)
  tile_used = (tidx < used_tiles).astype(jnp.int32)
  last_e = jnp.sum(used_tiles - 1 >= tile_end, axis=-1)
  tile_e = jnp.where(tile_used == 1, tile_e, last_e).astype(jnp.int32)

  tok_pad = jnp.zeros((PG,), jnp.int32).at[dest].set(t_flat)
  w_pad = jnp.zeros((P,), jnp.float32).at[dest].set(w_flat).reshape(P, 1)
  return dest, tok_pad, w_pad, tile_e, tile_used


def _sc_row_gather(table, idx, n_out, chunks):
  rows_w = n_out // NW
  rows_ch = rows_w // chunks
  tail = table.shape[1:]
  mesh = plsc.VectorSubcoreMesh(core_axis_name="c", subcore_axis_name="s")

  @functools.partial(
      pl.kernel,
      out_type=jax.ShapeDtypeStruct((n_out,) + tail, table.dtype),
      mesh=mesh,
      scratch_types=[
          pltpu.VMEM((rows_w,), jnp.int32),
          pltpu.VMEM((rows_ch,) + tail, table.dtype),
          pltpu.VMEM((rows_ch,) + tail, table.dtype),
          pltpu.SemaphoreType.DMA,
          pltpu.SemaphoreType.DMA,
      ],
  )
  def gather_k(table_hbm, idx_hbm, out_hbm, idx_v, rows_a, rows_b, sem_a, sem_b):
    wid = lax.axis_index("s") * 2 + lax.axis_index("c")
    base = wid * rows_w
    pltpu.sync_copy(idx_hbm.at[pl.ds(base, rows_w)], idx_v)
    bufs = [(rows_a, sem_a), (rows_b, sem_b)]
    cps = []
    for j in range(chunks):
      r, s = bufs[j % 2]
      cps.append(pltpu.async_copy(
          table_hbm.at[idx_v.at[pl.ds(j * rows_ch, rows_ch)]], r, s))
      if j >= 1:
        pr, _ = bufs[(j - 1) % 2]
        cps[j - 1].wait()
        pltpu.sync_copy(pr, out_hbm.at[pl.ds(base + (j - 1) * rows_ch, rows_ch)])
    cps[-1].wait()
    pltpu.sync_copy(bufs[(chunks - 1) % 2][0],
                    out_hbm.at[pl.ds(base + (chunks - 1) * rows_ch, rows_ch)])

  return gather_k(table, idx)


def _gemm_body(te_ref, tu_ref, xs_ref, w_ref, g_ref, u_ref, d_ref, y_ref):
  i = pl.program_id(0)

  @pl.when(tu_ref[i] == 1)
  def _():
    xb = xs_ref[...]
    dn = (((1,), (1,)), ((), ()))
    g = lax.dot_general(xb, g_ref[0], dn, preferred_element_type=jnp.float32)
    u = lax.dot_general(xb, u_ref[0], dn, preferred_element_type=jnp.float32)
    h = g * jax.nn.sigmoid(g) * u
    y = lax.dot_general(h, d_ref[0], dn, preferred_element_type=jnp.float32)
    y_ref[...] = y * w_ref[...]


def _grouped_gemm(xs, w_pad, gate_proj, up_proj, down_proj, tile_e, tile_used):
  grid_spec = pltpu.PrefetchScalarGridSpec(
      num_scalar_prefetch=2,
      grid=(R,),
      in_specs=[
          pl.BlockSpec((TM, D), lambda i, te, tu: (i, 0)),
          pl.BlockSpec((TM, 1), lambda i, te, tu: (i, 0)),
          pl.BlockSpec((1, F, D), lambda i, te, tu: (te[i], 0, 0)),
          pl.BlockSpec((1, F, D), lambda i, te, tu: (te[i], 0, 0)),
          pl.BlockSpec((1, D, F), lambda i, te, tu: (te[i], 0, 0)),
      ],
      out_specs=pl.BlockSpec((TM, D), lambda i, te, tu: (i, 0)),
  )
  return pl.pallas_call(
      _gemm_body,
      grid_spec=grid_spec,
      out_shape=jax.ShapeDtypeStruct((P, D), jnp.float32),
  )(tile_e, tile_used, xs, w_pad, gate_proj, up_proj, down_proj)


def _pair_sum_body(y2_ref, o_ref):
  o_ref[...] = y2_ref[:, :D] + y2_ref[:, D:]


def _pair_sum(y2):
  return pl.pallas_call(
      _pair_sum_body,
      grid=(T // 256,),
      in_specs=[pl.BlockSpec((256, K * D), lambda i: (i, 0))],
      out_specs=pl.BlockSpec((256, D), lambda i: (i, 0)),
      out_shape=jax.ShapeDtypeStruct((T, D), jnp.float32),
  )(y2)


def _sc_combine(y, dest):
  tok_w = T // NW
  tok_ch = 16
  n_ch = tok_w // tok_ch
  mesh = plsc.VectorSubcoreMesh(core_axis_name="c", subcore_axis_name="s")

  @functools.partial(
      pl.kernel,
      out_type=jax.ShapeDtypeStruct((T, 8, 128), jnp.float32),
      mesh=mesh,
      scratch_types=[
          pltpu.VMEM((K * tok_w,), jnp.int32),
          pltpu.VMEM((K * tok_ch, D), jnp.float32),
          pltpu.VMEM((K * tok_ch, D), jnp.float32),
          pltpu.VMEM((tok_ch, 8, 128), jnp.float32),
          pltpu.VMEM((tok_ch, 8, 128), jnp.float32),
          pltpu.SemaphoreType.DMA,
          pltpu.SemaphoreType.DMA,
          pltpu.SemaphoreType.DMA,
          pltpu.SemaphoreType.DMA,
      ],
  )
  def comb_k(y_hbm, dest_hbm, out_hbm, idx_v, ra, rb, oa, ob,
             gs_a, gs_b, ws_a, ws_b):
    wid = lax.axis_index("s") * 2 + lax.axis_index("c")
    base_s = wid * (K * tok_w)
    base_t = wid * tok_w
    pltpu.sync_copy(dest_hbm.at[pl.ds(base_s, K * tok_w)], idx_v)
    bufs = [(ra, gs_a, oa, ws_a), (rb, gs_b, ob, ws_b)]
    gat = [None] * n_ch
    wr = [None] * n_ch

    def process(j):
      r, _, obuf, ws = bufs[j % 2]
      gat[j].wait()

      def body(tk, c):
        for s in range(8):
          for q in range(8):
            off = s * 128 + q * 16
            a = r[2 * tk, pl.ds(off, 16)] + r[2 * tk + 1, pl.ds(off, 16)]
            obuf[tk, s, pl.ds(q * 16, 16)] = a
        return c

      lax.fori_loop(0, tok_ch, body, 0)
      wr[j] = pltpu.async_copy(
          obuf, out_hbm.at[pl.ds(base_t + j * tok_ch, tok_ch)], ws)

    for j in range(n_ch):
      r, gs, _, _ = bufs[j % 2]
      if j >= 2:
        wr[j - 2].wait()
      gat[j] = pltpu.async_copy(
          y_hbm.at[idx_v.at[pl.ds(j * K * tok_ch, K * tok_ch)]], r, gs)
      if j >= 1:
        process(j - 1)
    process(n_ch - 1)
    wr[n_ch - 2].wait()
    wr[n_ch - 1].wait()

  return comb_k(y, dest)


def kernel(x, gating_output, gate_proj, up_proj, down_proj):
  dest, tok_pad, w_pad, tile_e, tile_used = _routing_meta(gating_output)
  x3 = x.reshape(T, 8, 128)
  xs = _sc_row_gather(x3, tok_pad, PG, CH_D).reshape(PG, D)
  y = _grouped_gemm(xs, w_pad, gate_proj, up_proj, down_proj,
                    tile_e, tile_used)
  y2 = _sc_row_gather(y, dest, S, CH_C)
  return _pair_sum(y2.reshape(T, K * D))


# speedup vs baseline: 1.8631x; 1.3806x over previous
import functools

import jax
import jax.numpy as jnp
from jax import lax
from jax.experimental import pallas as pl
from jax.experimental.pallas import tpu as pltpu, tpu_sc as plsc

T = 2048
D = 1024
F = 2048
E = 8
K = 2
S = T * K
TM = 256
R = S // TM + E - 1
P = R * TM

NW = 32
PG = 6144
CH_D = 6
CH_C = 4


def _routing_meta(gating_output):
  probs = jax.nn.softmax(gating_output.astype(jnp.float32), axis=-1)
  topw, topi = lax.top_k(probs, K)
  topw = topw / jnp.sum(topw, axis=-1, keepdims=True)

  e_flat = topi.reshape(-1).astype(jnp.int32)
  w_flat = topw.reshape(-1)
  t_flat = (jnp.arange(S, dtype=jnp.int32) // K)

  onehot = (e_flat[:, None] == jnp.arange(E, dtype=jnp.int32)[None, :])
  onehot = onehot.astype(jnp.int32)
  cum = jnp.cumsum(onehot, axis=0)
  rank = jnp.take_along_axis(cum - onehot, e_flat[:, None], axis=1)[:, 0]
  sizes = cum[-1]
  tiles_e = (sizes + TM - 1) // TM
  tile_end = jnp.cumsum(tiles_e)
  pad_start = (tile_end - tiles_e) * TM

  dest = (pad_start[e_flat] + rank).astype(jnp.int32)

  used_tiles = tile_end[-1]
  tidx = jnp.arange(R, dtype=jnp.int32)
  tile_e = jnp.sum(tidx[:, None] >= tile_end[None, :], axis=-1)
  tile_used = (tidx < used_tiles).astype(jnp.int32)
  last_e = jnp.sum(used_tiles - 1 >= tile_end, axis=-1)
  tile_e = jnp.where(tile_used == 1, tile_e, last_e).astype(jnp.int32)

  tok_pad = (jnp.arange(PG, dtype=jnp.int32) % T).at[dest].set(t_flat)
  w_pad = jnp.zeros((P,), jnp.float32).at[dest].set(w_flat).reshape(P, 1)
  return dest, tok_pad, w_pad, tile_e, tile_used


def _sc_row_gather(table, idx, n_out, chunks):
  rows_w = n_out // NW
  rows_ch = rows_w // chunks
  tail = table.shape[1:]
  mesh = plsc.VectorSubcoreMesh(core_axis_name="c", subcore_axis_name="s")

  @functools.partial(
      pl.kernel,
      out_type=jax.ShapeDtypeStruct((n_out,) + tail, table.dtype),
      mesh=mesh,
      scratch_types=[
          pltpu.VMEM((rows_w,), jnp.int32),
          pltpu.VMEM((rows_ch,) + tail, table.dtype),
          pltpu.VMEM((rows_ch,) + tail, table.dtype),
          pltpu.SemaphoreType.DMA,
          pltpu.SemaphoreType.DMA,
      ],
  )
  def gather_k(table_hbm, idx_hbm, out_hbm, idx_v, rows_a, rows_b, sem_a, sem_b):
    wid = lax.axis_index("s") * 2 + lax.axis_index("c")
    base = wid * rows_w
    pltpu.sync_copy(idx_hbm.at[pl.ds(base, rows_w)], idx_v)
    bufs = [(rows_a, sem_a), (rows_b, sem_b)]
    cps = []
    for j in range(chunks):
      r, s = bufs[j % 2]
      cps.append(pltpu.async_copy(
          table_hbm.at[idx_v.at[pl.ds(j * rows_ch, rows_ch)]], r, s))
      if j >= 1:
        pr, _ = bufs[(j - 1) % 2]
        cps[j - 1].wait()
        pltpu.sync_copy(pr, out_hbm.at[pl.ds(base + (j - 1) * rows_ch, rows_ch)])
    cps[-1].wait()
    pltpu.sync_copy(bufs[(chunks - 1) % 2][0],
                    out_hbm.at[pl.ds(base + (chunks - 1) * rows_ch, rows_ch)])

  return gather_k(table, idx)


def _gemm_body(te_ref, tu_ref, xs_ref, w_ref, g_ref, u_ref, d_ref, y_ref):
  i = pl.program_id(0)

  @pl.when(tu_ref[i] == 1)
  def _():
    xb = xs_ref[...]
    dn = (((1,), (1,)), ((), ()))
    g = lax.dot_general(xb, g_ref[0], dn, preferred_element_type=jnp.float32)
    u = lax.dot_general(xb, u_ref[0], dn, preferred_element_type=jnp.float32)
    h = g * jax.nn.sigmoid(g) * u
    y = lax.dot_general(h, d_ref[0], dn, preferred_element_type=jnp.float32)
    y_ref[...] = y * w_ref[...]


def _grouped_gemm(xs, w_pad, gate_proj, up_proj, down_proj, tile_e, tile_used):
  grid_spec = pltpu.PrefetchScalarGridSpec(
      num_scalar_prefetch=2,
      grid=(R,),
      in_specs=[
          pl.BlockSpec((TM, D), lambda i, te, tu: (i, 0)),
          pl.BlockSpec((TM, 1), lambda i, te, tu: (i, 0)),
          pl.BlockSpec((1, F, D), lambda i, te, tu: (te[i], 0, 0)),
          pl.BlockSpec((1, F, D), lambda i, te, tu: (te[i], 0, 0)),
          pl.BlockSpec((1, D, F), lambda i, te, tu: (te[i], 0, 0)),
      ],
      out_specs=pl.BlockSpec((TM, D), lambda i, te, tu: (i, 0)),
  )
  return pl.pallas_call(
      _gemm_body,
      grid_spec=grid_spec,
      out_shape=jax.ShapeDtypeStruct((P, D), jnp.float32),
  )(tile_e, tile_used, xs, w_pad, gate_proj, up_proj, down_proj)


def _pair_sum_body(y2_ref, o_ref):
  o_ref[...] = y2_ref[:, :D] + y2_ref[:, D:]


def _pair_sum(y2):
  return pl.pallas_call(
      _pair_sum_body,
      grid=(T // 256,),
      in_specs=[pl.BlockSpec((256, K * D), lambda i: (i, 0))],
      out_specs=pl.BlockSpec((256, D), lambda i: (i, 0)),
      out_shape=jax.ShapeDtypeStruct((T, D), jnp.float32),
  )(y2)


def _sc_combine(y, dest):
  tok_w = T // NW
  tok_ch = 16
  n_ch = tok_w // tok_ch
  mesh = plsc.VectorSubcoreMesh(core_axis_name="c", subcore_axis_name="s")

  @functools.partial(
      pl.kernel,
      out_type=jax.ShapeDtypeStruct((T, 8, 128), jnp.float32),
      mesh=mesh,
      scratch_types=[
          pltpu.VMEM((K * tok_w,), jnp.int32),
          pltpu.VMEM((K * tok_ch, D), jnp.float32),
          pltpu.VMEM((K * tok_ch, D), jnp.float32),
          pltpu.VMEM((tok_ch, 8, 128), jnp.float32),
          pltpu.VMEM((tok_ch, 8, 128), jnp.float32),
          pltpu.SemaphoreType.DMA,
          pltpu.SemaphoreType.DMA,
          pltpu.SemaphoreType.DMA,
          pltpu.SemaphoreType.DMA,
      ],
  )
  def comb_k(y_hbm, dest_hbm, out_hbm, idx_v, ra, rb, oa, ob,
             gs_a, gs_b, ws_a, ws_b):
    wid = lax.axis_index("s") * 2 + lax.axis_index("c")
    base_s = wid * (K * tok_w)
    base_t = wid * tok_w
    pltpu.sync_copy(dest_hbm.at[pl.ds(base_s, K * tok_w)], idx_v)
    bufs = [(ra, gs_a, oa, ws_a), (rb, gs_b, ob, ws_b)]
    gat = [None] * n_ch
    wr = [None] * n_ch

    def process(j):
      r, _, obuf, ws = bufs[j % 2]
      gat[j].wait()

      def body(tk, c):
        for s in range(8):
          for q in range(8):
            off = s * 128 + q * 16
            a = r[2 * tk, pl.ds(off, 16)] + r[2 * tk + 1, pl.ds(off, 16)]
            obuf[tk, s, pl.ds(q * 16, 16)] = a
        return c

      lax.fori_loop(0, tok_ch, body, 0)
      wr[j] = pltpu.async_copy(
          obuf, out_hbm.at[pl.ds(base_t + j * tok_ch, tok_ch)], ws)

    for j in range(n_ch):
      r, gs, _, _ = bufs[j % 2]
      if j >= 2:
        wr[j - 2].wait()
      gat[j] = pltpu.async_copy(
          y_hbm.at[idx_v.at[pl.ds(j * K * tok_ch, K * tok_ch)]], r, gs)
      if j >= 1:
        process(j - 1)
    process(n_ch - 1)
    wr[n_ch - 2].wait()
    wr[n_ch - 1].wait()

  return comb_k(y, dest)


def kernel(x, gating_output, gate_proj, up_proj, down_proj):
  dest, tok_pad, w_pad, tile_e, tile_used = _routing_meta(gating_output)
  x3 = x.reshape(T, 8, 128)
  xs = _sc_row_gather(x3, tok_pad, PG, CH_D).reshape(PG, D)
  y = _grouped_gemm(xs, w_pad, gate_proj, up_proj, down_proj,
                    tile_e, tile_used)
  y2 = _sc_row_gather(y, dest, S, CH_C)
  return _pair_sum(y2.reshape(T, K * D))


# speedup vs baseline: 2.0506x; 1.1006x over previous
import functools

import jax
import jax.numpy as jnp
from jax import lax
from jax.experimental import pallas as pl
from jax.experimental.pallas import tpu as pltpu, tpu_sc as plsc

T = 2048
D = 1024
F = 2048
E = 8
K = 2
S = T * K
TM = 256
R = S // TM + E - 1
P = R * TM

NW = 32
PG = 6144
CH_D = 6
CH_C = 4


def _routing_meta(gating_output):
  probs = jax.nn.softmax(gating_output.astype(jnp.float32), axis=-1)
  topw, topi = lax.top_k(probs, K)
  topw = topw / jnp.sum(topw, axis=-1, keepdims=True)

  e_flat = topi.reshape(-1).astype(jnp.int32)
  w_flat = topw.reshape(-1)
  t_flat = (jnp.arange(S, dtype=jnp.int32) // K)

  onehot = (e_flat[:, None] == jnp.arange(E, dtype=jnp.int32)[None, :])
  onehot = onehot.astype(jnp.int32)
  cum = jnp.cumsum(onehot, axis=0)
  rank = jnp.take_along_axis(cum - onehot, e_flat[:, None], axis=1)[:, 0]
  sizes = cum[-1]
  tiles_e = (sizes + TM - 1) // TM
  tile_end = jnp.cumsum(tiles_e)
  pad_start = (tile_end - tiles_e) * TM

  dest = (pad_start[e_flat] + rank).astype(jnp.int32)

  used_tiles = tile_end[-1]
  tidx = jnp.arange(R, dtype=jnp.int32)
  tile_e = jnp.sum(tidx[:, None] >= tile_end[None, :], axis=-1)
  tile_used = (tidx < used_tiles).astype(jnp.int32)
  last_e = jnp.sum(used_tiles - 1 >= tile_end, axis=-1)
  tile_e = jnp.where(tile_used == 1, tile_e, last_e).astype(jnp.int32)

  tok_pad = (jnp.arange(PG, dtype=jnp.int32) % T).at[dest].set(t_flat)
  w_pad = jnp.zeros((P,), jnp.float32).at[dest].set(w_flat).reshape(P, 1)
  return dest, tok_pad, w_pad, tile_e, tile_used


def _sc_row_gather(table, idx, n_out, chunks):
  rows_w = n_out // NW
  rows_ch = rows_w // chunks
  tail = table.shape[1:]
  mesh = plsc.VectorSubcoreMesh(core_axis_name="c", subcore_axis_name="s")

  @functools.partial(
      pl.kernel,
      out_type=jax.ShapeDtypeStruct((n_out,) + tail, table.dtype),
      mesh=mesh,
      scratch_types=[
          pltpu.VMEM((rows_w,), jnp.int32),
          pltpu.VMEM((rows_ch,) + tail, table.dtype),
          pltpu.VMEM((rows_ch,) + tail, table.dtype),
          pltpu.SemaphoreType.DMA,
          pltpu.SemaphoreType.DMA,
      ],
  )
  def gather_k(table_hbm, idx_hbm, out_hbm, idx_v, rows_a, rows_b, sem_a, sem_b):
    wid = lax.axis_index("s") * 2 + lax.axis_index("c")
    base = wid * rows_w
    pltpu.sync_copy(idx_hbm.at[pl.ds(base, rows_w)], idx_v)
    bufs = [(rows_a, sem_a), (rows_b, sem_b)]
    cps = []
    for j in range(chunks):
      r, s = bufs[j % 2]
      cps.append(pltpu.async_copy(
          table_hbm.at[idx_v.at[pl.ds(j * rows_ch, rows_ch)]], r, s))
      if j >= 1:
        pr, _ = bufs[(j - 1) % 2]
        cps[j - 1].wait()
        pltpu.sync_copy(pr, out_hbm.at[pl.ds(base + (j - 1) * rows_ch, rows_ch)])
    cps[-1].wait()
    pltpu.sync_copy(bufs[(chunks - 1) % 2][0],
                    out_hbm.at[pl.ds(base + (chunks - 1) * rows_ch, rows_ch)])

  return gather_k(table, idx)


def _gemm_body(te_ref, tu_ref, xs_ref, w_ref, g_ref, u_ref, d_ref, y_ref):
  i = pl.program_id(0)

  @pl.when(tu_ref[i] == 1)
  def _():
    xb = xs_ref[...]
    dn = (((1,), (1,)), ((), ()))
    g = lax.dot_general(xb, g_ref[0], dn, preferred_element_type=jnp.float32)
    u = lax.dot_general(xb, u_ref[0], dn, preferred_element_type=jnp.float32)
    h = g * jax.nn.sigmoid(g) * u
    y = lax.dot_general(h, d_ref[0], dn, preferred_element_type=jnp.float32)
    y_ref[...] = y * w_ref[...]


def _grouped_gemm(xs, w_pad, gate_proj, up_proj, down_proj, tile_e, tile_used):
  grid_spec = pltpu.PrefetchScalarGridSpec(
      num_scalar_prefetch=2,
      grid=(R,),
      in_specs=[
          pl.BlockSpec((TM, D), lambda i, te, tu: (i, 0)),
          pl.BlockSpec((TM, 1), lambda i, te, tu: (i, 0)),
          pl.BlockSpec((1, F, D), lambda i, te, tu: (te[i], 0, 0)),
          pl.BlockSpec((1, F, D), lambda i, te, tu: (te[i], 0, 0)),
          pl.BlockSpec((1, D, F), lambda i, te, tu: (te[i], 0, 0)),
      ],
      out_specs=pl.BlockSpec((TM, D), lambda i, te, tu: (i, 0)),
  )
  return pl.pallas_call(
      _gemm_body,
      grid_spec=grid_spec,
      out_shape=jax.ShapeDtypeStruct((P, D), jnp.float32),
  )(tile_e, tile_used, xs, w_pad, gate_proj, up_proj, down_proj)


def _pair_sum_body(y2_ref, o_ref):
  o_ref[...] = y2_ref[:, :D] + y2_ref[:, D:]


def _pair_sum(y2):
  return pl.pallas_call(
      _pair_sum_body,
      grid=(T // 256,),
      in_specs=[pl.BlockSpec((256, K * D), lambda i: (i, 0))],
      out_specs=pl.BlockSpec((256, D), lambda i: (i, 0)),
      out_shape=jax.ShapeDtypeStruct((T, D), jnp.float32),
  )(y2)


def _sc_combine(y, dest):
  tok_w = T // NW
  tok_ch = 16
  n_ch = tok_w // tok_ch
  mesh = plsc.VectorSubcoreMesh(core_axis_name="c", subcore_axis_name="s")

  @functools.partial(
      pl.kernel,
      out_type=jax.ShapeDtypeStruct((T, 8, 128), jnp.float32),
      mesh=mesh,
      scratch_types=[
          pltpu.VMEM((K * tok_w,), jnp.int32),
          pltpu.VMEM((K * tok_ch, D), jnp.float32),
          pltpu.VMEM((K * tok_ch, D), jnp.float32),
          pltpu.VMEM((tok_ch, 8, 128), jnp.float32),
          pltpu.VMEM((tok_ch, 8, 128), jnp.float32),
          pltpu.SemaphoreType.DMA,
          pltpu.SemaphoreType.DMA,
          pltpu.SemaphoreType.DMA,
          pltpu.SemaphoreType.DMA,
      ],
  )
  def comb_k(y_hbm, dest_hbm, out_hbm, idx_v, ra, rb, oa, ob,
             gs_a, gs_b, ws_a, ws_b):
    wid = lax.axis_index("s") * 2 + lax.axis_index("c")
    base_s = wid * (K * tok_w)
    base_t = wid * tok_w
    pltpu.sync_copy(dest_hbm.at[pl.ds(base_s, K * tok_w)], idx_v)
    bufs = [(ra, gs_a, oa, ws_a), (rb, gs_b, ob, ws_b)]
    gat = [None] * n_ch
    wr = [None] * n_ch

    def process(j):
      r, _, obuf, ws = bufs[j % 2]
      gat[j].wait()

      def body(tk, c):
        for s in range(8):
          for q in range(8):
            off = s * 128 + q * 16
            a = r[2 * tk, pl.ds(off, 16)] + r[2 * tk + 1, pl.ds(off, 16)]
            obuf[tk, s, pl.ds(q * 16, 16)] = a
        return c

      lax.fori_loop(0, tok_ch, body, 0)
      wr[j] = pltpu.async_copy(
          obuf, out_hbm.at[pl.ds(base_t + j * tok_ch, tok_ch)], ws)

    for j in range(n_ch):
      r, gs, _, _ = bufs[j % 2]
      if j >= 2:
        wr[j - 2].wait()
      gat[j] = pltpu.async_copy(
          y_hbm.at[idx_v.at[pl.ds(j * K * tok_ch, K * tok_ch)]], r, gs)
      if j >= 1:
        process(j - 1)
    process(n_ch - 1)
    wr[n_ch - 2].wait()
    wr[n_ch - 1].wait()

  return comb_k(y, dest)


def kernel(x, gating_output, gate_proj, up_proj, down_proj):
  dest, tok_pad, w_pad, tile_e, tile_used = _routing_meta(gating_output)
  xs = _sc_row_gather(x, tok_pad, PG, CH_D)
  y = _grouped_gemm(xs, w_pad, gate_proj, up_proj, down_proj,
                    tile_e, tile_used)
  y2 = _sc_row_gather(y, dest, S, CH_C)
  return _pair_sum(y2.reshape(T, K * D))


# speedup vs baseline: 2.0804x; 1.0145x over previous
import functools

import jax
import jax.numpy as jnp
from jax import lax
from jax.experimental import pallas as pl
from jax.experimental.pallas import tpu as pltpu, tpu_sc as plsc

T = 2048
D = 1024
F = 2048
E = 8
K = 2
S = T * K
TM = 256
R = S // TM + E - 1
P = R * TM

NW = 32
PG = 6144
CH_D = 6
CH_C = 4


def _routing_meta(gating_output):
  probs = jax.nn.softmax(gating_output.astype(jnp.float32), axis=-1)
  topw, topi = lax.top_k(probs, K)
  topw = topw / jnp.sum(topw, axis=-1, keepdims=True)

  e_flat = topi.reshape(-1).astype(jnp.int32)
  w_flat = topw.reshape(-1)
  t_flat = (jnp.arange(S, dtype=jnp.int32) // K)

  onehot = (e_flat[:, None] == jnp.arange(E, dtype=jnp.int32)[None, :])
  onehot = onehot.astype(jnp.int32)
  cum = jnp.cumsum(onehot, axis=0)
  rank = jnp.take_along_axis(cum - onehot, e_flat[:, None], axis=1)[:, 0]
  sizes = cum[-1]
  tiles_e = (sizes + TM - 1) // TM
  tile_end = jnp.cumsum(tiles_e)
  pad_start = (tile_end - tiles_e) * TM

  dest = (pad_start[e_flat] + rank).astype(jnp.int32)

  used_tiles = tile_end[-1]
  tidx = jnp.arange(R, dtype=jnp.int32)
  tile_e = jnp.sum(tidx[:, None] >= tile_end[None, :], axis=-1)
  tile_used = (tidx < used_tiles).astype(jnp.int32)
  last_e = jnp.sum(used_tiles - 1 >= tile_end, axis=-1)
  tile_e = jnp.where(tile_used == 1, tile_e, last_e).astype(jnp.int32)

  tok_pad = (jnp.arange(PG, dtype=jnp.int32) % T).at[dest].set(t_flat)
  w_pad = jnp.zeros((P,), jnp.float32).at[dest].set(w_flat).reshape(P, 1)
  return dest, tok_pad, w_pad, tile_e, tile_used


def _sc_row_gather(table, idx, n_out, chunks):
  rows_w = n_out // NW
  rows_ch = rows_w // chunks
  tail = table.shape[1:]
  mesh = plsc.VectorSubcoreMesh(core_axis_name="c", subcore_axis_name="s")

  @functools.partial(
      pl.kernel,
      out_type=jax.ShapeDtypeStruct((n_out,) + tail, table.dtype),
      mesh=mesh,
      scratch_types=[
          pltpu.VMEM((rows_w,), jnp.int32),
          pltpu.VMEM((rows_ch,) + tail, table.dtype),
          pltpu.VMEM((rows_ch,) + tail, table.dtype),
          pltpu.SemaphoreType.DMA,
          pltpu.SemaphoreType.DMA,
      ],
  )
  def gather_k(table_hbm, idx_hbm, out_hbm, idx_v, rows_a, rows_b, sem_a, sem_b):
    wid = lax.axis_index("s") * 2 + lax.axis_index("c")
    base = wid * rows_w
    pltpu.sync_copy(idx_hbm.at[pl.ds(base, rows_w)], idx_v)
    bufs = [(rows_a, sem_a), (rows_b, sem_b)]
    cps = []
    for j in range(chunks):
      r, s = bufs[j % 2]
      cps.append(pltpu.async_copy(
          table_hbm.at[idx_v.at[pl.ds(j * rows_ch, rows_ch)]], r, s))
      if j >= 1:
        pr, _ = bufs[(j - 1) % 2]
        cps[j - 1].wait()
        pltpu.sync_copy(pr, out_hbm.at[pl.ds(base + (j - 1) * rows_ch, rows_ch)])
    cps[-1].wait()
    pltpu.sync_copy(bufs[(chunks - 1) % 2][0],
                    out_hbm.at[pl.ds(base + (chunks - 1) * rows_ch, rows_ch)])

  return gather_k(table, idx)


def _gemm_body(te_ref, tu_ref, xs_ref, w_ref, g_ref, u_ref, d_ref, y_ref):
  i = pl.program_id(0)

  @pl.when(tu_ref[i] == 1)
  def _():
    xb = xs_ref[...]
    dn = (((1,), (1,)), ((), ()))
    g = lax.dot_general(xb, g_ref[0], dn, preferred_element_type=jnp.float32)
    u = lax.dot_general(xb, u_ref[0], dn, preferred_element_type=jnp.float32)
    h = g * jax.nn.sigmoid(g) * u
    y = lax.dot_general(h, d_ref[0], dn, preferred_element_type=jnp.float32)
    y_ref[...] = y * w_ref[...]


def _grouped_gemm(xs, w_pad, gate_proj, up_proj, down_proj, tile_e, tile_used):
  grid_spec = pltpu.PrefetchScalarGridSpec(
      num_scalar_prefetch=2,
      grid=(R,),
      in_specs=[
          pl.BlockSpec((TM, D), lambda i, te, tu: (i, 0)),
          pl.BlockSpec((TM, 1), lambda i, te, tu: (i, 0)),
          pl.BlockSpec((1, F, D), lambda i, te, tu: (te[i], 0, 0)),
          pl.BlockSpec((1, F, D), lambda i, te, tu: (te[i], 0, 0)),
          pl.BlockSpec((1, D, F), lambda i, te, tu: (te[i], 0, 0)),
      ],
      out_specs=pl.BlockSpec((TM, D), lambda i, te, tu: (i, 0)),
  )
  return pl.pallas_call(
      _gemm_body,
      grid_spec=grid_spec,
      out_shape=jax.ShapeDtypeStruct((P, D), jnp.float32),
  )(tile_e, tile_used, xs, w_pad, gate_proj, up_proj, down_proj)


def _pair_sum_body(y2_ref, o_ref):
  o_ref[...] = y2_ref[:, :D] + y2_ref[:, D:]


def _pair_sum(y2):
  return pl.pallas_call(
      _pair_sum_body,
      grid=(T // 256,),
      in_specs=[pl.BlockSpec((256, K * D), lambda i: (i, 0))],
      out_specs=pl.BlockSpec((256, D), lambda i: (i, 0)),
      out_shape=jax.ShapeDtypeStruct((T, D), jnp.float32),
  )(y2)


def _sc_combine(y, dest):
  tok_w = T // NW
  tok_ch = 16
  n_ch = tok_w // tok_ch
  mesh = plsc.VectorSubcoreMesh(core_axis_name="c", subcore_axis_name="s")

  @functools.partial(
      pl.kernel,
      out_type=jax.ShapeDtypeStruct((T, 8, 128), jnp.float32),
      mesh=mesh,
      scratch_types=[
          pltpu.VMEM((K * tok_w,), jnp.int32),
          pltpu.VMEM((K * tok_ch, D), jnp.float32),
          pltpu.VMEM((K * tok_ch, D), jnp.float32),
          pltpu.VMEM((tok_ch, 8, 128), jnp.float32),
          pltpu.VMEM((tok_ch, 8, 128), jnp.float32),
          pltpu.SemaphoreType.DMA,
          pltpu.SemaphoreType.DMA,
          pltpu.SemaphoreType.DMA,
          pltpu.SemaphoreType.DMA,
      ],
  )
  def comb_k(y_hbm, dest_hbm, out_hbm, idx_v, ra, rb, oa, ob,
             gs_a, gs_b, ws_a, ws_b):
    wid = lax.axis_index("s") * 2 + lax.axis_index("c")
    base_s = wid * (K * tok_w)
    base_t = wid * tok_w
    pltpu.sync_copy(dest_hbm.at[pl.ds(base_s, K * tok_w)], idx_v)
    bufs = [(ra, gs_a, oa, ws_a), (rb, gs_b, ob, ws_b)]
    gat = [None] * n_ch
    wr = [None] * n_ch

    def process(j):
      r, _, obuf, ws = bufs[j % 2]
      gat[j].wait()

      def body(tk, c):
        for s in range(8):
          for q in range(8):
            off = s * 128 + q * 16
            a = r[2 * tk, pl.ds(off, 16)] + r[2 * tk + 1, pl.ds(off, 16)]
            obuf[tk, s, pl.ds(q * 16, 16)] = a
        return c

      lax.fori_loop(0, tok_ch, body, 0)
      wr[j] = pltpu.async_copy(
          obuf, out_hbm.at[pl.ds(base_t + j * tok_ch, tok_ch)], ws)

    for j in range(n_ch):
      r, gs, _, _ = bufs[j % 2]
      if j >= 2:
        wr[j - 2].wait()
      gat[j] = pltpu.async_copy(
          y_hbm.at[idx_v.at[pl.ds(j * K * tok_ch, K * tok_ch)]], r, gs)
      if j >= 1:
        process(j - 1)
    process(n_ch - 1)
    wr[n_ch - 2].wait()
    wr[n_ch - 1].wait()

  return comb_k(y, dest)


def kernel(x, gating_output, gate_proj, up_proj, down_proj):
  dest, tok_pad, w_pad, tile_e, tile_used = _routing_meta(gating_output)
  xs = _sc_row_gather(x, tok_pad, PG, CH_D)
  y = _grouped_gemm(xs, w_pad, gate_proj, up_proj, down_proj,
                    tile_e, tile_used)
  return _sc_combine(y, dest).reshape(T, D)
